# baseline (speedup 1.0000x reference)
import jax
import jax.numpy as jnp
from jax import lax
from jax.experimental import pallas as pl
from jax.experimental.pallas import tpu as pltpu
from jax.experimental.pallas import tpu_sc as plsc

_B = 32
_N = 16384
_L = 16
_RB = 8
_C = 2048
_NSH = _N // _C

_NEG_INF = float("-inf")


def _combine(a, b):
    take = (b[0] > a[0]) | ((b[0] == a[0]) & (b[1] < a[1]))
    return tuple(jnp.where(take, y, x) for x, y in zip(a, b))


def _butterfly(acc, lanes):
    for sh in (8, 4, 2, 1):
        perm = lanes ^ sh
        other = tuple(v.at[perm].get(mode="promise_in_bounds") for v in acc)
        acc = _combine(acc, other)
    return acc


def _fps_body(xyz_hbm, out_hbm, x_v, y_v, z_v,
              pub_v, comb_v, stage_v, stage_o,
              sem_x, sem_y, sem_z):
    c = lax.axis_index("c")
    s = lax.axis_index("s")
    slab = s // _NSH
    shard = s % _NSH
    row0 = c * 16 + slab * _RB
    col0 = shard * _C

    lanes = lax.iota(jnp.int32, _L)

    cp_y = pltpu.async_copy(
        xyz_hbm.at[0, 1, pl.ds(row0, _RB), pl.ds(col0, _C)], y_v, sem_y)
    cp_x = pltpu.async_copy(
        xyz_hbm.at[0, 0, pl.ds(row0, _RB), pl.ds(col0, _C)], x_v, sem_x)
    cp_z = pltpu.async_copy(
        xyz_hbm.at[0, 2, pl.ds(row0, _RB), pl.ds(col0, _C)], z_v, sem_z)

    cp_y.wait()
    cp_x.wait()
    cp_z.wait()

    init1 = tuple(
        (
            jnp.full((_L,), _NEG_INF, jnp.float32),
            jnp.zeros((_L,), jnp.int32),
            jnp.zeros((_L,), jnp.float32),
            jnp.zeros((_L,), jnp.float32),
        )
        for _ in range(_RB)
    )

    @plsc.parallel_loop(0, _C, step=_L, unroll=2, carry=init1)
    def accs1(i, accs):
        idx = lanes + (col0 + i)
        sl = pl.ds(i, _L)
        out = []
        for k in range(_RB):
            av, ai, ax, az = accs[k]
            v = y_v[k, sl]
            take = v > av
            out.append((
                jnp.where(take, v, av),
                jnp.where(take, idx, ai),
                jnp.where(take, x_v[k, sl], ax),
                jnp.where(take, z_v[k, sl], az),
            ))
        return tuple(out)

    for k in range(_RB):
        v, i, x, z = accs1[k]
        stage_v[pl.ds(k * 64, _L)] = v
        stage_v[pl.ds(k * 64 + 16, _L)] = plsc.bitcast(i, jnp.float32)
        stage_v[pl.ds(k * 64 + 32, _L)] = x
        stage_v[pl.ds(k * 64 + 48, _L)] = z
    pltpu.sync_copy(stage_v, pub_v.at[pl.ds(s * 512, 512)])
    plsc.subcore_barrier()

    pltpu.sync_copy(pub_v.at[pl.ds(slab * _NSH * 512, _NSH * 512)], comb_v)

    def _load1(r, k):
        base = r * 512 + k * 64
        return (comb_v[pl.ds(base, _L)],
                plsc.bitcast(comb_v[pl.ds(base + 16, _L)], jnp.int32),
                comb_v[pl.ds(base + 32, _L)],
                comb_v[pl.ds(base + 48, _L)])

    cents = []
    for k in range(_RB):
        acc = _load1(0, k)
        for r in range(1, _NSH):
            acc = _combine(acc, _load1(r, k))
        cents.append(_butterfly(acc, lanes))
    plsc.subcore_barrier()

    init2 = tuple(
        (
            jnp.full((_L,), _NEG_INF, jnp.float32),
            jnp.zeros((_L,), jnp.int32),
        )
        for _ in range(_RB)
    )

    @plsc.parallel_loop(0, _C, step=_L, unroll=2, carry=init2)
    def accs2(i, accs):
        idx = lanes + (col0 + i)
        sl = pl.ds(i, _L)
        out = []
        for k in range(_RB):
            cy, _, cx, cz = cents[k]
            av, ai = accs[k]
            dx = x_v[k, sl] - cx
            dy = y_v[k, sl] - cy
            dz = z_v[k, sl] - cz
            d = dx * dx + dy * dy + dz * dz
            d = jnp.minimum(d, jnp.float32(1e10))
            take = d > av
            out.append((jnp.where(take, d, av), jnp.where(take, idx, ai)))
        return tuple(out)

    for k in range(_RB):
        v, i = accs2[k]
        stage_v[pl.ds(k * 64, _L)] = v
        stage_v[pl.ds(k * 64 + 16, _L)] = plsc.bitcast(i, jnp.float32)
    pltpu.sync_copy(stage_v, pub_v.at[pl.ds(s * 512, 512)])
    plsc.subcore_barrier()

    pltpu.sync_copy(pub_v.at[pl.ds(slab * _NSH * 512, _NSH * 512)], comb_v)

    def _load2(r, k):
        base = r * 512 + k * 64
        return (comb_v[pl.ds(base, _L)],
                plsc.bitcast(comb_v[pl.ds(base + 16, _L)], jnp.int32))

    @pl.when(shard == 0)
    def _():
        for k in range(_RB):
            acc = _load2(0, k)
            for r in range(1, _NSH):
                acc = _combine(acc, _load2(r, k))
            _, i1 = _butterfly(acc, lanes)
            i0 = cents[k][1]
            stage_o[...] = jnp.where(lanes == 0, i0, i1)
            pltpu.sync_copy(stage_o, out_hbm.at[row0 + k])


@jax.jit
def kernel(xyz):
    pts = jnp.transpose(xyz, (0, 2, 1, 3))
    mesh = plsc.VectorSubcoreMesh(core_axis_name="c", subcore_axis_name="s")
    fps = pl.kernel(
        _fps_body,
        out_type=jax.ShapeDtypeStruct((_B, _L), jnp.int32),
        mesh=mesh,
        compiler_params=pltpu.CompilerParams(
            use_tc_tiling_on_sc=True, needs_layout_passes=False
        ),
        scratch_types=(
            [pltpu.VMEM((_RB, _C), jnp.float32) for _ in range(3)]
            + [
                pltpu.VMEM_SHARED((16 * 512,), jnp.float32),
                pltpu.VMEM((_NSH * 512,), jnp.float32),
                pltpu.VMEM((512,), jnp.float32),
                pltpu.VMEM((_L,), jnp.int32),
            ]
            + [pltpu.SemaphoreType.DMA for _ in range(3)]
        ),
    )
    out = fps(pts)
    return out[:, :2]


# speedup vs baseline: 1.0467x; 1.0467x over previous
import jax
import jax.numpy as jnp
from jax import lax
from jax.experimental import pallas as pl
from jax.experimental.pallas import tpu as pltpu
from jax.experimental.pallas import tpu_sc as plsc

_B = 32
_N = 16384
_L = 16
_RB = 8
_C = 2048
_NSH = _N // _C

_NEG_INF = float("-inf")


def _combine(a, b):
    take = (b[0] > a[0]) | ((b[0] == a[0]) & (b[1] < a[1]))
    return tuple(jnp.where(take, y, x) for x, y in zip(a, b))


def _butterfly(acc, lanes):
    for sh in (8, 4, 2, 1):
        perm = lanes ^ sh
        other = tuple(v.at[perm].get(mode="promise_in_bounds") for v in acc)
        acc = _combine(acc, other)
    return acc


def _fps_body(xyz_hbm, out_hbm, x_v, y_v, z_v,
              pub_v, comb_v, stage_v, cent_v, stage_o,
              sem_x, sem_y, sem_z):
    c = lax.axis_index("c")
    s = lax.axis_index("s")
    slab = s // _NSH
    shard = s % _NSH
    row0 = c * 16 + slab * _RB
    col0 = shard * _C

    lanes = lax.iota(jnp.int32, _L)

    cp_y = pltpu.async_copy(
        xyz_hbm.at[0, 1, pl.ds(row0, _RB), pl.ds(col0, _C)], y_v, sem_y)
    cp_x = pltpu.async_copy(
        xyz_hbm.at[0, 0, pl.ds(row0, _RB), pl.ds(col0, _C)], x_v, sem_x)
    cp_z = pltpu.async_copy(
        xyz_hbm.at[0, 2, pl.ds(row0, _RB), pl.ds(col0, _C)], z_v, sem_z)

    cp_y.wait()
    cp_x.wait()
    cp_z.wait()

    init1 = tuple(
        (
            jnp.full((_L,), _NEG_INF, jnp.float32),
            jnp.zeros((_L,), jnp.int32),
            jnp.zeros((_L,), jnp.float32),
            jnp.zeros((_L,), jnp.float32),
        )
        for _ in range(_RB)
    )

    @plsc.parallel_loop(0, _C, step=_L, carry=init1)
    def accs1(i, accs):
        idx = lanes + (col0 + i)
        sl = pl.ds(i, _L)
        out = []
        for k in range(_RB):
            av, ai, ax, az = accs[k]
            v = y_v[k, sl]
            take = v > av
            out.append((
                jnp.where(take, v, av),
                jnp.where(take, idx, ai),
                jnp.where(take, x_v[k, sl], ax),
                jnp.where(take, z_v[k, sl], az),
            ))
        return tuple(out)

    for k in range(_RB):
        v, i, x, z = accs1[k]
        stage_v[pl.ds(k * 64, _L)] = v
        stage_v[pl.ds(k * 64 + 16, _L)] = plsc.bitcast(i, jnp.float32)
        stage_v[pl.ds(k * 64 + 32, _L)] = x
        stage_v[pl.ds(k * 64 + 48, _L)] = z
    pltpu.sync_copy(stage_v, pub_v.at[pl.ds(s * 512, 512)])
    plsc.subcore_barrier()

    pltpu.sync_copy(pub_v.at[pl.ds(slab * _NSH * 512, _NSH * 512)], comb_v)

    @plsc.parallel_loop(0, _RB)
    def _merge1(k):
        def load(r):
            base = r * 512 + k * 64
            return (comb_v[pl.ds(base, _L)],
                    plsc.bitcast(comb_v[pl.ds(base + 16, _L)], jnp.int32),
                    comb_v[pl.ds(base + 32, _L)],
                    comb_v[pl.ds(base + 48, _L)])

        acc = load(0)
        for r in range(1, _NSH):
            acc = _combine(acc, load(r))
        cy, i0, cx, cz = _butterfly(acc, lanes)
        cent_v[pl.ds(k * 64, _L)] = cy
        cent_v[pl.ds(k * 64 + 16, _L)] = plsc.bitcast(i0, jnp.float32)
        cent_v[pl.ds(k * 64 + 32, _L)] = cx
        cent_v[pl.ds(k * 64 + 48, _L)] = cz

    cents = [
        (cent_v[pl.ds(k * 64, _L)],
         plsc.bitcast(cent_v[pl.ds(k * 64 + 16, _L)], jnp.int32),
         cent_v[pl.ds(k * 64 + 32, _L)],
         cent_v[pl.ds(k * 64 + 48, _L)])
        for k in range(_RB)
    ]

    init2 = tuple(
        (
            jnp.full((_L,), _NEG_INF, jnp.float32),
            jnp.zeros((_L,), jnp.int32),
        )
        for _ in range(_RB)
    )

    @plsc.parallel_loop(0, _C, step=_L, carry=init2)
    def accs2(i, accs):
        idx = lanes + (col0 + i)
        sl = pl.ds(i, _L)
        out = []
        for k in range(_RB):
            cy, _, cx, cz = cents[k]
            av, ai = accs[k]
            dx = x_v[k, sl] - cx
            dy = y_v[k, sl] - cy
            dz = z_v[k, sl] - cz
            d = dx * dx + dy * dy + dz * dz
            d = jnp.minimum(d, jnp.float32(1e10))
            take = d > av
            out.append((jnp.where(take, d, av), jnp.where(take, idx, ai)))
        return tuple(out)

    for k in range(_RB):
        v, i = accs2[k]
        stage_v[pl.ds(k * 64, _L)] = v
        stage_v[pl.ds(k * 64 + 16, _L)] = plsc.bitcast(i, jnp.float32)
    pltpu.sync_copy(stage_v, pub_v.at[pl.ds(8192 + s * 512, 512)])
    plsc.subcore_barrier()

    @pl.when(shard == 0)
    def _():
        pltpu.sync_copy(
            pub_v.at[pl.ds(8192 + slab * _NSH * 512, _NSH * 512)], comb_v)

        @plsc.parallel_loop(0, _RB)
        def _merge2(k):
            def load(r):
                base = r * 512 + k * 64
                return (comb_v[pl.ds(base, _L)],
                        plsc.bitcast(comb_v[pl.ds(base + 16, _L)], jnp.int32))

            acc = load(0)
            for r in range(1, _NSH):
                acc = _combine(acc, load(r))
            _, i1 = _butterfly(acc, lanes)
            i0 = plsc.bitcast(cent_v[pl.ds(k * 64 + 16, _L)], jnp.int32)
            pair = jnp.where(lanes == 0, i0, i1)
            stage_v[pl.ds(k * 64, _L)] = plsc.bitcast(pair, jnp.float32)

        for k in range(_RB):
            stage_o[...] = plsc.bitcast(stage_v[pl.ds(k * 64, _L)], jnp.int32)
            pltpu.sync_copy(stage_o, out_hbm.at[row0 + k])


@jax.jit
def kernel(xyz):
    pts = jnp.transpose(xyz, (0, 2, 1, 3))
    mesh = plsc.VectorSubcoreMesh(core_axis_name="c", subcore_axis_name="s")
    fps = pl.kernel(
        _fps_body,
        out_type=jax.ShapeDtypeStruct((_B, _L), jnp.int32),
        mesh=mesh,
        compiler_params=pltpu.CompilerParams(
            use_tc_tiling_on_sc=True, needs_layout_passes=False
        ),
        scratch_types=(
            [pltpu.VMEM((_RB, _C), jnp.float32) for _ in range(3)]
            + [
                pltpu.VMEM_SHARED((2 * 16 * 512,), jnp.float32),
                pltpu.VMEM((_NSH * 512,), jnp.float32),
                pltpu.VMEM((512,), jnp.float32),
                pltpu.VMEM((512,), jnp.float32),
                pltpu.VMEM((_L,), jnp.int32),
            ]
            + [pltpu.SemaphoreType.DMA for _ in range(3)]
        ),
    )
    out = fps(pts)
    return out[:, :2]


# speedup vs baseline: 1.0840x; 1.0357x over previous
import jax
import jax.numpy as jnp
from jax import lax
from jax.experimental import pallas as pl
from jax.experimental.pallas import tpu as pltpu
from jax.experimental.pallas import tpu_sc as plsc

_B = 32
_N = 16384
_L = 16
_RB = 8
_C = 2048
_NSH = _N // _C

_NEG_INF = float("-inf")


def _combine(a, b):
    take = (b[0] > a[0]) | ((b[0] == a[0]) & (b[1] < a[1]))
    return tuple(jnp.where(take, y, x) for x, y in zip(a, b))


def _butterfly(acc, lanes):
    for sh in (8, 4, 2, 1):
        perm = lanes ^ sh
        other = tuple(v.at[perm].get(mode="promise_in_bounds") for v in acc)
        acc = _combine(acc, other)
    return acc


def _fps_body(xyz_hbm, out_hbm, x_v, y_v, z_v,
              pub_v, comb_v, stage_v, cent_v, stage_o,
              sem_x, sem_y, sem_z):
    c = lax.axis_index("c")
    s = lax.axis_index("s")
    slab = s // _NSH
    shard = s % _NSH
    row0 = c * 16 + slab * _RB
    col0 = shard * _C

    lanes = lax.iota(jnp.int32, _L)

    cp_y = pltpu.async_copy(
        xyz_hbm.at[0, 1, pl.ds(row0, _RB), pl.ds(col0, _C)], y_v, sem_y)
    cp_x = pltpu.async_copy(
        xyz_hbm.at[0, 0, pl.ds(row0, _RB), pl.ds(col0, _C)], x_v, sem_x)
    cp_z = pltpu.async_copy(
        xyz_hbm.at[0, 2, pl.ds(row0, _RB), pl.ds(col0, _C)], z_v, sem_z)

    cp_y.wait()

    init1 = tuple(
        (
            jnp.full((_L,), _NEG_INF, jnp.float32),
            jnp.zeros((_L,), jnp.int32),
        )
        for _ in range(_RB)
    )

    @plsc.parallel_loop(0, _C, step=_L, carry=init1)
    def accs1(i, accs):
        idx = lanes + (col0 + i)
        sl = pl.ds(i, _L)
        out = []
        for k in range(_RB):
            av, ai = accs[k]
            v = y_v[k, sl]
            take = v > av
            out.append((jnp.where(take, v, av), jnp.where(take, idx, ai)))
        return tuple(out)

    cp_x.wait()
    cp_z.wait()
    for k in range(_RB):
        v, i = _butterfly(accs1[k], lanes)
        loc = i & (_C - 1)
        rowk = jnp.full((_L,), k, jnp.int32)
        x = plsc.load_gather(x_v, [rowk, loc])
        z = plsc.load_gather(z_v, [rowk, loc])
        stage_v[pl.ds(k * 64, _L)] = v
        stage_v[pl.ds(k * 64 + 16, _L)] = plsc.bitcast(i, jnp.float32)
        stage_v[pl.ds(k * 64 + 32, _L)] = x
        stage_v[pl.ds(k * 64 + 48, _L)] = z
    pltpu.sync_copy(stage_v, pub_v.at[pl.ds(s * 512, 512)])
    plsc.subcore_barrier()

    pltpu.sync_copy(pub_v.at[pl.ds(slab * _NSH * 512, _NSH * 512)], comb_v)

    @plsc.parallel_loop(0, _RB)
    def _merge1(k):
        def load(r):
            base = r * 512 + k * 64
            return (comb_v[pl.ds(base, _L)],
                    plsc.bitcast(comb_v[pl.ds(base + 16, _L)], jnp.int32),
                    comb_v[pl.ds(base + 32, _L)],
                    comb_v[pl.ds(base + 48, _L)])

        acc = load(0)
        for r in range(1, _NSH):
            acc = _combine(acc, load(r))
        cy, i0, cx, cz = acc
        cent_v[pl.ds(k * 64, _L)] = cy
        cent_v[pl.ds(k * 64 + 16, _L)] = plsc.bitcast(i0, jnp.float32)
        cent_v[pl.ds(k * 64 + 32, _L)] = cx
        cent_v[pl.ds(k * 64 + 48, _L)] = cz

    cents = [
        (cent_v[pl.ds(k * 64, _L)],
         plsc.bitcast(cent_v[pl.ds(k * 64 + 16, _L)], jnp.int32),
         cent_v[pl.ds(k * 64 + 32, _L)],
         cent_v[pl.ds(k * 64 + 48, _L)])
        for k in range(_RB)
    ]

    init2 = tuple(
        (
            jnp.full((_L,), _NEG_INF, jnp.float32),
            jnp.zeros((_L,), jnp.int32),
        )
        for _ in range(_RB)
    )

    @plsc.parallel_loop(0, _C, step=_L, carry=init2)
    def accs2(i, accs):
        idx = lanes + (col0 + i)
        sl = pl.ds(i, _L)
        out = []
        for k in range(_RB):
            cy, _, cx, cz = cents[k]
            av, ai = accs[k]
            dx = x_v[k, sl] - cx
            dy = y_v[k, sl] - cy
            dz = z_v[k, sl] - cz
            d = dx * dx + dy * dy + dz * dz
            d = jnp.minimum(d, jnp.float32(1e10))
            take = d > av
            out.append((jnp.where(take, d, av), jnp.where(take, idx, ai)))
        return tuple(out)

    for k in range(_RB):
        v, i = _butterfly(accs2[k], lanes)
        stage_v[pl.ds(k * 64, _L)] = v
        stage_v[pl.ds(k * 64 + 16, _L)] = plsc.bitcast(i, jnp.float32)
    pltpu.sync_copy(stage_v, pub_v.at[pl.ds(8192 + s * 512, 512)])
    plsc.subcore_barrier()

    @pl.when(shard == 0)
    def _():
        pltpu.sync_copy(
            pub_v.at[pl.ds(8192 + slab * _NSH * 512, _NSH * 512)], comb_v)

        @plsc.parallel_loop(0, _RB)
        def _merge2(k):
            def load(r):
                base = r * 512 + k * 64
                return (comb_v[pl.ds(base, _L)],
                        plsc.bitcast(comb_v[pl.ds(base + 16, _L)], jnp.int32))

            acc = load(0)
            for r in range(1, _NSH):
                acc = _combine(acc, load(r))
            _, i1 = acc
            i0 = plsc.bitcast(cent_v[pl.ds(k * 64 + 16, _L)], jnp.int32)
            pair = jnp.where(lanes == 0, i0, i1)
            stage_v[pl.ds(k * 64, _L)] = plsc.bitcast(pair, jnp.float32)

        for k in range(_RB):
            stage_o[...] = plsc.bitcast(stage_v[pl.ds(k * 64, _L)], jnp.int32)
            pltpu.sync_copy(stage_o, out_hbm.at[row0 + k])


@jax.jit
def kernel(xyz):
    pts = jnp.transpose(xyz, (0, 2, 1, 3))
    mesh = plsc.VectorSubcoreMesh(core_axis_name="c", subcore_axis_name="s")
    fps = pl.kernel(
        _fps_body,
        out_type=jax.ShapeDtypeStruct((_B, _L), jnp.int32),
        mesh=mesh,
        compiler_params=pltpu.CompilerParams(
            use_tc_tiling_on_sc=True, needs_layout_passes=False
        ),
        scratch_types=(
            [pltpu.VMEM((_RB, _C), jnp.float32) for _ in range(3)]
            + [
                pltpu.VMEM_SHARED((2 * 16 * 512,), jnp.float32),
                pltpu.VMEM((_NSH * 512,), jnp.float32),
                pltpu.VMEM((512,), jnp.float32),
                pltpu.VMEM((512,), jnp.float32),
                pltpu.VMEM((_L,), jnp.int32),
            ]
            + [pltpu.SemaphoreType.DMA for _ in range(3)]
        ),
    )
    out = fps(pts)
    return out[:, :2]
